# baseline (speedup 1.0000x reference)
import functools

import jax
import jax.numpy as jnp
from jax import lax
from jax.experimental import pallas as pl
from jax.experimental.pallas import tpu as pltpu
from jax.experimental.pallas import tpu_sc as plsc

N_TOK = 8192
D_MODEL = 1024
D_FF = 2048
N_EXPERTS = 16
TOP_K = 2
CAPACITY = 1280
EC = N_EXPERTS * CAPACITY
DUMMY = EC
EC_PAD = EC + 8

ROUTER_B = 512
NW = 32
TOK_PER_W = N_TOK // NW
CHUNK = 32
C_TILE = 640
N_CT = CAPACITY // C_TILE


def _router_body(x_ref, wg_ref, ridx_ref, gate_ref, cnt_ref):
    B = ROUTER_B
    E = N_EXPERTS
    pi = pl.program_id(0)

    @pl.when(pi == 0)
    def _():
        cnt_ref[...] = jnp.zeros_like(cnt_ref)

    logits = jnp.dot(x_ref[...], wg_ref[...],
                     preferred_element_type=jnp.float32)
    iota = lax.broadcasted_iota(jnp.int32, (B, E), 1)
    m1 = jnp.max(logits, axis=1, keepdims=True)
    i1 = jnp.min(jnp.where(logits == m1, iota, E), axis=1, keepdims=True)
    masked = jnp.where(iota == i1, -jnp.inf, logits)
    m2 = jnp.max(masked, axis=1, keepdims=True)
    i2 = jnp.min(jnp.where(masked == m2, iota, E), axis=1, keepdims=True)

    t = jnp.exp(m2 - m1)
    g0 = 1.0 / (1.0 + t)
    g1 = t / (1.0 + t)

    oh0 = (iota == i1).astype(jnp.float32)
    oh1 = (iota == i2).astype(jnp.float32)
    oh = oh0 + oh1
    tri = (lax.broadcasted_iota(jnp.int32, (B, B), 0) >
           lax.broadcasted_iota(jnp.int32, (B, B), 1)).astype(jnp.float32)
    excl = jnp.dot(tri, oh, preferred_element_type=jnp.float32)
    sx = cnt_ref[...] + excl
    pos0 = jnp.sum(sx * oh0, axis=1, keepdims=True).astype(jnp.int32)
    pos1 = jnp.sum(sx * oh1, axis=1, keepdims=True).astype(jnp.int32)
    cnt_ref[...] += jnp.sum(oh, axis=0, keepdims=True)

    r0 = jnp.where(pos0 < CAPACITY, i1 * CAPACITY + pos0, DUMMY)
    r1 = jnp.where(pos1 < CAPACITY, i2 * CAPACITY + pos1, DUMMY)
    ridx_ref[:, 0:1] = r0
    ridx_ref[:, 1:2] = r1
    gate_ref[:, 0:1] = g0
    gate_ref[:, 1:2] = g1


def _router(x, Wg):
    nblk = N_TOK // ROUTER_B
    return pl.pallas_call(
        _router_body,
        grid=(nblk,),
        in_specs=[
            pl.BlockSpec((ROUTER_B, D_MODEL), lambda i: (i, 0)),
            pl.BlockSpec((D_MODEL, N_EXPERTS), lambda i: (0, 0)),
        ],
        out_specs=[
            pl.BlockSpec((ROUTER_B, 2), lambda i: (i, 0)),
            pl.BlockSpec((ROUTER_B, 2), lambda i: (i, 0)),
        ],
        out_shape=[
            jax.ShapeDtypeStruct((N_TOK, 2), jnp.int32),
            jax.ShapeDtypeStruct((N_TOK, 2), jnp.float32),
        ],
        scratch_shapes=[pltpu.VMEM((1, N_EXPERTS), jnp.float32)],
    )(x, Wg)


N_CH = TOK_PER_W // CHUNK


def _dispatch(x, ridx4):
    mesh = plsc.VectorSubcoreMesh(core_axis_name="c", subcore_axis_name="s")

    @functools.partial(
        pl.kernel,
        mesh=mesh,
        out_type=jax.ShapeDtypeStruct((EC_PAD, D_MODEL), jnp.float32),
        scratch_types=[
            pltpu.VMEM((CHUNK, D_MODEL), jnp.float32),
            pltpu.VMEM((CHUNK, D_MODEL), jnp.float32),
            pltpu.VMEM((N_CH, CHUNK), jnp.int32),
            pltpu.VMEM((N_CH, CHUNK), jnp.int32),
            pltpu.SemaphoreType.DMA,
            pltpu.SemaphoreType.DMA,
            pltpu.SemaphoreType.DMA,
            pltpu.SemaphoreType.DMA,
        ],
    )
    def dispatch_kernel(x_hbm, ridx4_hbm, xbuf_hbm, xv0, xv1, iv0, iv1,
                        l0, l1, s0, s1):
        wid = lax.axis_index("s") * 2 + lax.axis_index("c")
        base = wid * TOK_PER_W
        pltpu.sync_copy(ridx4_hbm.at[0, wid], iv0)
        pltpu.sync_copy(ridx4_hbm.at[1, wid], iv1)
        xvs = (xv0, xv1)
        lsems = (l0, l1)
        ssems = (s0, s1)
        loads = {}
        scats = {}
        loads[0] = pltpu.async_copy(x_hbm.at[pl.ds(base, CHUNK)], xv0, l0)
        for k in range(N_CH):
            b = k % 2
            if k + 1 < N_CH:
                nb = (k + 1) % 2
                if k - 1 >= 0:
                    scats[k - 1][0].wait()
                    scats[k - 1][1].wait()
                loads[k + 1] = pltpu.async_copy(
                    x_hbm.at[pl.ds(base + (k + 1) * CHUNK, CHUNK)],
                    xvs[nb], lsems[nb])
            loads[k].wait()
            c0 = pltpu.async_copy(xvs[b], xbuf_hbm.at[iv0.at[k]], ssems[b])
            c1 = pltpu.async_copy(xvs[b], xbuf_hbm.at[iv1.at[k]], ssems[b])
            scats[k] = (c0, c1)
        for k in (N_CH - 2, N_CH - 1):
            scats[k][0].wait()
            scats[k][1].wait()

    return dispatch_kernel(x, ridx4)


F_TILE = 1024
N_FT = D_FF // F_TILE


def _ffn_body(x_ref, w1_ref, w2_ref, y_ref, acc_ref):
    f = pl.program_id(1)
    h = jnp.maximum(
        jnp.dot(x_ref[...], w1_ref[0], preferred_element_type=jnp.float32),
        0.0)
    part = jnp.dot(h, w2_ref[0], preferred_element_type=jnp.float32)

    @pl.when(f == 0)
    def _():
        acc_ref[...] = part

    @pl.when(f != 0)
    def _():
        y_ref[...] = (acc_ref[...] + part).astype(jnp.bfloat16)


def _ffn(xbuf, W1, W2):
    return pl.pallas_call(
        _ffn_body,
        grid=(N_EXPERTS, N_FT),
        in_specs=[
            pl.BlockSpec((CAPACITY, D_MODEL), lambda e, f: (e, 0)),
            pl.BlockSpec((1, D_MODEL, F_TILE), lambda e, f: (e, 0, f)),
            pl.BlockSpec((1, F_TILE, D_MODEL), lambda e, f: (e, f, 0)),
        ],
        out_specs=pl.BlockSpec((CAPACITY, D_MODEL), lambda e, f: (e, 0)),
        out_shape=jax.ShapeDtypeStruct((EC_PAD, D_MODEL), jnp.bfloat16),
        scratch_shapes=[pltpu.VMEM((CAPACITY, D_MODEL), jnp.float32)],
        compiler_params=pltpu.CompilerParams(
            dimension_semantics=("parallel", "arbitrary")),
    )(xbuf, W1, W2)


CG = 16
N_CHG = TOK_PER_W // CG


D_PK = D_MODEL // 2


def _combine_gather(y32, gidx):
    mesh = plsc.VectorSubcoreMesh(core_axis_name="c", subcore_axis_name="s")

    @functools.partial(
        pl.kernel,
        mesh=mesh,
        out_type=jax.ShapeDtypeStruct((2, N_TOK, D_PK), jnp.int32),
        scratch_types=[
            pltpu.VMEM((2 * CG, D_PK), jnp.int32),
            pltpu.VMEM((2 * CG, D_PK), jnp.int32),
            pltpu.VMEM((N_CHG, 2 * CG), jnp.int32),
            pltpu.SemaphoreType.DMA,
            pltpu.SemaphoreType.DMA,
            pltpu.SemaphoreType.DMA,
            pltpu.SemaphoreType.DMA,
        ],
    )
    def gather_kernel(y_hbm, gidx_hbm, yg_hbm, yv0, yv1, iv, g0, g1, w0, w1):
        wid = lax.axis_index("s") * 2 + lax.axis_index("c")
        base = wid * TOK_PER_W
        pltpu.sync_copy(gidx_hbm.at[wid], iv)
        yvs = (yv0, yv1)
        gsems = (g0, g1)
        wsems = (w0, w1)
        gath = {}
        wr = {}
        gath[0] = pltpu.async_copy(y_hbm.at[iv.at[0]], yv0, g0)
        for k in range(N_CHG):
            b = k % 2
            if k + 1 < N_CHG:
                nb = (k + 1) % 2
                if k - 1 >= 0:
                    wr[k - 1][0].wait()
                    wr[k - 1][1].wait()
                gath[k + 1] = pltpu.async_copy(
                    y_hbm.at[iv.at[k + 1]], yvs[nb], gsems[nb])
            gath[k].wait()
            t0 = base + k * CG
            c0 = pltpu.async_copy(yvs[b].at[pl.ds(0, CG)],
                                  yg_hbm.at[0, pl.ds(t0, CG)], wsems[b])
            c1 = pltpu.async_copy(yvs[b].at[pl.ds(CG, CG)],
                                  yg_hbm.at[1, pl.ds(t0, CG)], wsems[b])
            wr[k] = (c0, c1)
        for k in (N_CHG - 2, N_CHG - 1):
            wr[k][0].wait()
            wr[k][1].wait()

    return gather_kernel(y32, gidx)


def _combine_body(yg_ref, ridx_ref, gate_ref, o_ref):
    v0 = ridx_ref[:, 0:1] != DUMMY
    v1 = ridx_ref[:, 1:2] != DUMMY
    a0 = jnp.where(v0, gate_ref[:, 0:1] * yg_ref[0].astype(jnp.float32), 0.0)
    a1 = jnp.where(v1, gate_ref[:, 1:2] * yg_ref[1].astype(jnp.float32), 0.0)
    o_ref[...] = a0 + a1


def _combine(yg, ridx, gates):
    B = ROUTER_B
    nblk = N_TOK // B
    return pl.pallas_call(
        _combine_body,
        grid=(nblk,),
        in_specs=[
            pl.BlockSpec((2, B, D_MODEL), lambda i: (0, i, 0)),
            pl.BlockSpec((B, 2), lambda i: (i, 0)),
            pl.BlockSpec((B, 2), lambda i: (i, 0)),
        ],
        out_specs=pl.BlockSpec((B, D_MODEL), lambda i: (i, 0)),
        out_shape=jax.ShapeDtypeStruct((N_TOK, D_MODEL), jnp.float32),
        compiler_params=pltpu.CompilerParams(
            dimension_semantics=("parallel",)),
    )(yg, ridx, gates)


def kernel(x, Wg, W1, W2):
    ridx, gates = _router(x, Wg)
    ridx_sc = ridx.T
    ridx4 = ridx_sc.reshape(2, NW, N_CH, CHUNK)
    gidx = (ridx_sc.reshape(2, NW, N_CHG, CG)
            .transpose(1, 2, 0, 3).reshape(NW, N_CHG, 2 * CG))
    xbuf = _dispatch(x, ridx4)
    y = _ffn(xbuf, W1, W2)
    y32 = jax.lax.bitcast_convert_type(
        y.reshape(EC_PAD, D_PK, 2), jnp.int32)
    yg32 = _combine_gather(y32, gidx)
    yg = jax.lax.bitcast_convert_type(
        yg32, jnp.bfloat16).reshape(2, N_TOK, D_MODEL)
    return _combine(yg, ridx, gates)


# speedup vs baseline: 3.2738x; 3.2738x over previous
import functools

import jax
import jax.numpy as jnp
from jax import lax
from jax.experimental import pallas as pl
from jax.experimental.pallas import tpu as pltpu
from jax.experimental.pallas import tpu_sc as plsc

N_TOK = 8192
D_MODEL = 1024
D_FF = 2048
N_EXPERTS = 16
TOP_K = 2
CAPACITY = 1280
EC = N_EXPERTS * CAPACITY
DUMMY = EC
EC_PAD = EC + 8

ROUTER_B = 512
NW = 32
TOK_PER_W = N_TOK // NW
CHUNK = 32
C_TILE = 640
N_CT = CAPACITY // C_TILE


def _router_body(x_ref, wg_ref, ridx_ref, gate_ref, cnt_ref):
    B = ROUTER_B
    E = N_EXPERTS
    pi = pl.program_id(0)

    @pl.when(pi == 0)
    def _():
        cnt_ref[...] = jnp.zeros_like(cnt_ref)

    logits = jnp.dot(x_ref[...], wg_ref[...],
                     preferred_element_type=jnp.float32)
    iota = lax.broadcasted_iota(jnp.int32, (B, E), 1)
    m1 = jnp.max(logits, axis=1, keepdims=True)
    i1 = jnp.min(jnp.where(logits == m1, iota, E), axis=1, keepdims=True)
    masked = jnp.where(iota == i1, -jnp.inf, logits)
    m2 = jnp.max(masked, axis=1, keepdims=True)
    i2 = jnp.min(jnp.where(masked == m2, iota, E), axis=1, keepdims=True)

    t = jnp.exp(m2 - m1)
    g0 = 1.0 / (1.0 + t)
    g1 = t / (1.0 + t)

    oh0 = (iota == i1).astype(jnp.float32)
    oh1 = (iota == i2).astype(jnp.float32)
    oh = oh0 + oh1
    tri = (lax.broadcasted_iota(jnp.int32, (B, B), 0) >
           lax.broadcasted_iota(jnp.int32, (B, B), 1)).astype(jnp.float32)
    excl = jnp.dot(tri, oh, preferred_element_type=jnp.float32)
    sx = cnt_ref[...] + excl
    pos0 = jnp.sum(sx * oh0, axis=1, keepdims=True).astype(jnp.int32)
    pos1 = jnp.sum(sx * oh1, axis=1, keepdims=True).astype(jnp.int32)
    cnt_ref[...] += jnp.sum(oh, axis=0, keepdims=True)

    r0 = jnp.where(pos0 < CAPACITY, i1 * CAPACITY + pos0, DUMMY)
    r1 = jnp.where(pos1 < CAPACITY, i2 * CAPACITY + pos1, DUMMY)
    ridx_ref[:, 0:1] = r0
    ridx_ref[:, 1:2] = r1
    gate_ref[:, 0:1] = g0
    gate_ref[:, 1:2] = g1


def _router(x, Wg):
    nblk = N_TOK // ROUTER_B
    return pl.pallas_call(
        _router_body,
        grid=(nblk,),
        in_specs=[
            pl.BlockSpec((ROUTER_B, D_MODEL), lambda i: (i, 0)),
            pl.BlockSpec((D_MODEL, N_EXPERTS), lambda i: (0, 0)),
        ],
        out_specs=[
            pl.BlockSpec((ROUTER_B, 2), lambda i: (i, 0)),
            pl.BlockSpec((ROUTER_B, 2), lambda i: (i, 0)),
        ],
        out_shape=[
            jax.ShapeDtypeStruct((N_TOK, 2), jnp.int32),
            jax.ShapeDtypeStruct((N_TOK, 2), jnp.float32),
        ],
        scratch_shapes=[pltpu.VMEM((1, N_EXPERTS), jnp.float32)],
    )(x, Wg)


N_CH = TOK_PER_W // CHUNK


def _dispatch(x, ridx4):
    mesh = plsc.VectorSubcoreMesh(core_axis_name="c", subcore_axis_name="s")

    @functools.partial(
        pl.kernel,
        mesh=mesh,
        out_type=jax.ShapeDtypeStruct((EC_PAD, D_MODEL), jnp.float32),
        scratch_types=[
            pltpu.VMEM((CHUNK, D_MODEL), jnp.float32),
            pltpu.VMEM((CHUNK, D_MODEL), jnp.float32),
            pltpu.VMEM((N_CH, CHUNK), jnp.int32),
            pltpu.VMEM((N_CH, CHUNK), jnp.int32),
            pltpu.SemaphoreType.DMA,
            pltpu.SemaphoreType.DMA,
            pltpu.SemaphoreType.DMA,
            pltpu.SemaphoreType.DMA,
        ],
    )
    def dispatch_kernel(x_hbm, ridx4_hbm, xbuf_hbm, xv0, xv1, iv0, iv1,
                        l0, l1, s0, s1):
        wid = lax.axis_index("s") * 2 + lax.axis_index("c")
        base = wid * TOK_PER_W
        pltpu.sync_copy(ridx4_hbm.at[0, wid], iv0)
        pltpu.sync_copy(ridx4_hbm.at[1, wid], iv1)
        xvs = (xv0, xv1)
        lsems = (l0, l1)
        ssems = (s0, s1)
        loads = {}
        scats = {}
        loads[0] = pltpu.async_copy(x_hbm.at[pl.ds(base, CHUNK)], xv0, l0)
        for k in range(N_CH):
            b = k % 2
            if k + 1 < N_CH:
                nb = (k + 1) % 2
                if k - 1 >= 0:
                    scats[k - 1][0].wait()
                    scats[k - 1][1].wait()
                loads[k + 1] = pltpu.async_copy(
                    x_hbm.at[pl.ds(base + (k + 1) * CHUNK, CHUNK)],
                    xvs[nb], lsems[nb])
            loads[k].wait()
            c0 = pltpu.async_copy(xvs[b], xbuf_hbm.at[iv0.at[k]], ssems[b])
            c1 = pltpu.async_copy(xvs[b], xbuf_hbm.at[iv1.at[k]], ssems[b])
            scats[k] = (c0, c1)
        for k in (N_CH - 2, N_CH - 1):
            scats[k][0].wait()
            scats[k][1].wait()

    return dispatch_kernel(x, ridx4)


F_TILE = 1024
N_FT = D_FF // F_TILE
D_PK = D_MODEL // 2


def _ffn_body(x_ref, w1_ref, w2_ref, y_ref, acc_ref):
    f = pl.program_id(1)
    h = jnp.maximum(
        jnp.dot(x_ref[...], w1_ref[0], preferred_element_type=jnp.float32),
        0.0)
    part = jnp.dot(h, w2_ref[0], preferred_element_type=jnp.float32)

    @pl.when(f == 0)
    def _():
        acc_ref[...] = part

    @pl.when(f != 0)
    def _():
        yb = (acc_ref[...] + part).astype(jnp.bfloat16)
        hi = lax.bitcast_convert_type(yb[:, :D_PK], jnp.int16).astype(jnp.int32)
        lo = lax.bitcast_convert_type(yb[:, D_PK:], jnp.int16).astype(jnp.int32)
        y_ref[...] = (hi << 16) | (lo & 0xFFFF)


def _ffn(xbuf, W1, W2):
    return pl.pallas_call(
        _ffn_body,
        grid=(N_EXPERTS, N_FT),
        in_specs=[
            pl.BlockSpec((CAPACITY, D_MODEL), lambda e, f: (e, 0)),
            pl.BlockSpec((1, D_MODEL, F_TILE), lambda e, f: (e, 0, f)),
            pl.BlockSpec((1, F_TILE, D_MODEL), lambda e, f: (e, f, 0)),
        ],
        out_specs=pl.BlockSpec((CAPACITY, D_PK), lambda e, f: (e, 0)),
        out_shape=jax.ShapeDtypeStruct((EC_PAD, D_PK), jnp.int32),
        scratch_shapes=[pltpu.VMEM((CAPACITY, D_MODEL), jnp.float32)],
        compiler_params=pltpu.CompilerParams(
            dimension_semantics=("parallel", "arbitrary")),
    )(xbuf, W1, W2)


CG = 16
N_CHG = TOK_PER_W // CG


def _combine_gather(y32, gidx):
    mesh = plsc.VectorSubcoreMesh(core_axis_name="c", subcore_axis_name="s")

    @functools.partial(
        pl.kernel,
        mesh=mesh,
        out_type=jax.ShapeDtypeStruct((2, N_TOK, D_PK), jnp.int32),
        scratch_types=[
            pltpu.VMEM((2 * CG, D_PK), jnp.int32),
            pltpu.VMEM((2 * CG, D_PK), jnp.int32),
            pltpu.VMEM((N_CHG, 2 * CG), jnp.int32),
            pltpu.SemaphoreType.DMA,
            pltpu.SemaphoreType.DMA,
            pltpu.SemaphoreType.DMA,
            pltpu.SemaphoreType.DMA,
        ],
    )
    def gather_kernel(y_hbm, gidx_hbm, yg_hbm, yv0, yv1, iv, g0, g1, w0, w1):
        wid = lax.axis_index("s") * 2 + lax.axis_index("c")
        base = wid * TOK_PER_W
        pltpu.sync_copy(gidx_hbm.at[wid], iv)
        yvs = (yv0, yv1)
        gsems = (g0, g1)
        wsems = (w0, w1)
        gath = {}
        wr = {}
        gath[0] = pltpu.async_copy(y_hbm.at[iv.at[0]], yv0, g0)
        for k in range(N_CHG):
            b = k % 2
            if k + 1 < N_CHG:
                nb = (k + 1) % 2
                if k - 1 >= 0:
                    wr[k - 1][0].wait()
                    wr[k - 1][1].wait()
                gath[k + 1] = pltpu.async_copy(
                    y_hbm.at[iv.at[k + 1]], yvs[nb], gsems[nb])
            gath[k].wait()
            t0 = base + k * CG
            c0 = pltpu.async_copy(yvs[b].at[pl.ds(0, CG)],
                                  yg_hbm.at[0, pl.ds(t0, CG)], wsems[b])
            c1 = pltpu.async_copy(yvs[b].at[pl.ds(CG, CG)],
                                  yg_hbm.at[1, pl.ds(t0, CG)], wsems[b])
            wr[k] = (c0, c1)
        for k in (N_CHG - 2, N_CHG - 1):
            wr[k][0].wait()
            wr[k][1].wait()

    return gather_kernel(y32, gidx)


def _unpack_bf16(w):
    hi = lax.bitcast_convert_type(
        lax.shift_right_logical(w, 16).astype(jnp.int16), jnp.bfloat16)
    lo = lax.bitcast_convert_type(w.astype(jnp.int16), jnp.bfloat16)
    return jnp.concatenate(
        [hi.astype(jnp.float32), lo.astype(jnp.float32)], axis=1)


def _combine_body(yg_ref, ridx_ref, gate_ref, o_ref):
    v0 = ridx_ref[:, 0:1] != DUMMY
    v1 = ridx_ref[:, 1:2] != DUMMY
    a0 = jnp.where(v0, gate_ref[:, 0:1] * _unpack_bf16(yg_ref[0]), 0.0)
    a1 = jnp.where(v1, gate_ref[:, 1:2] * _unpack_bf16(yg_ref[1]), 0.0)
    o_ref[...] = a0 + a1


def _combine(yg, ridx, gates):
    B = ROUTER_B
    nblk = N_TOK // B
    return pl.pallas_call(
        _combine_body,
        grid=(nblk,),
        in_specs=[
            pl.BlockSpec((2, B, D_PK), lambda i: (0, i, 0)),
            pl.BlockSpec((B, 2), lambda i: (i, 0)),
            pl.BlockSpec((B, 2), lambda i: (i, 0)),
        ],
        out_specs=pl.BlockSpec((B, D_MODEL), lambda i: (i, 0)),
        out_shape=jax.ShapeDtypeStruct((N_TOK, D_MODEL), jnp.float32),
        compiler_params=pltpu.CompilerParams(
            dimension_semantics=("parallel",)),
    )(yg, ridx, gates)


def kernel(x, Wg, W1, W2):
    ridx, gates = _router(x, Wg)
    ridx_sc = ridx.T
    ridx4 = ridx_sc.reshape(2, NW, N_CH, CHUNK)
    gidx = (ridx_sc.reshape(2, NW, N_CHG, CG)
            .transpose(1, 2, 0, 3).reshape(NW, N_CHG, 2 * CG))
    xbuf = _dispatch(x, ridx4)
    y32 = _ffn(xbuf, W1, W2)
    yg32 = _combine_gather(y32, gidx)
    return _combine(yg32, ridx, gates)


# speedup vs baseline: 3.2808x; 1.0021x over previous
import functools

import jax
import jax.numpy as jnp
from jax import lax
from jax.experimental import pallas as pl
from jax.experimental.pallas import tpu as pltpu
from jax.experimental.pallas import tpu_sc as plsc

N_TOK = 8192
D_MODEL = 1024
D_FF = 2048
N_EXPERTS = 16
TOP_K = 2
CAPACITY = 1280
EC = N_EXPERTS * CAPACITY
DUMMY = EC
EC_PAD = EC + 8

ROUTER_B = 512
NW = 32
TOK_PER_W = N_TOK // NW
CHUNK = 32
C_TILE = 640
N_CT = CAPACITY // C_TILE


def _router_body(x_ref, wg_ref, ridx_ref, gate_ref, cnt_ref):
    B = ROUTER_B
    E = N_EXPERTS
    pi = pl.program_id(0)

    @pl.when(pi == 0)
    def _():
        cnt_ref[...] = jnp.zeros_like(cnt_ref)

    logits = jnp.dot(x_ref[...], wg_ref[...],
                     preferred_element_type=jnp.float32)
    iota = lax.broadcasted_iota(jnp.int32, (B, E), 1)
    m1 = jnp.max(logits, axis=1, keepdims=True)
    i1 = jnp.min(jnp.where(logits == m1, iota, E), axis=1, keepdims=True)
    masked = jnp.where(iota == i1, -jnp.inf, logits)
    m2 = jnp.max(masked, axis=1, keepdims=True)
    i2 = jnp.min(jnp.where(masked == m2, iota, E), axis=1, keepdims=True)

    t = jnp.exp(m2 - m1)
    g0 = 1.0 / (1.0 + t)
    g1 = t / (1.0 + t)

    oh0 = (iota == i1).astype(jnp.float32)
    oh1 = (iota == i2).astype(jnp.float32)
    oh = oh0 + oh1
    tri = (lax.broadcasted_iota(jnp.int32, (B, B), 0) >
           lax.broadcasted_iota(jnp.int32, (B, B), 1)).astype(jnp.float32)
    excl = jnp.dot(tri, oh, preferred_element_type=jnp.float32)
    sx = cnt_ref[...] + excl
    pos0 = jnp.sum(sx * oh0, axis=1, keepdims=True).astype(jnp.int32)
    pos1 = jnp.sum(sx * oh1, axis=1, keepdims=True).astype(jnp.int32)
    cnt_ref[...] += jnp.sum(oh, axis=0, keepdims=True)

    r0 = jnp.where(pos0 < CAPACITY, i1 * CAPACITY + pos0, DUMMY)
    r1 = jnp.where(pos1 < CAPACITY, i2 * CAPACITY + pos1, DUMMY)
    ridx_ref[:, 0:1] = r0
    ridx_ref[:, 1:2] = r1
    gate_ref[:, 0:1] = g0
    gate_ref[:, 1:2] = g1


def _router(x, Wg):
    nblk = N_TOK // ROUTER_B
    return pl.pallas_call(
        _router_body,
        grid=(nblk,),
        in_specs=[
            pl.BlockSpec((ROUTER_B, D_MODEL), lambda i: (i, 0)),
            pl.BlockSpec((D_MODEL, N_EXPERTS), lambda i: (0, 0)),
        ],
        out_specs=[
            pl.BlockSpec((ROUTER_B, 2), lambda i: (i, 0)),
            pl.BlockSpec((ROUTER_B, 2), lambda i: (i, 0)),
        ],
        out_shape=[
            jax.ShapeDtypeStruct((N_TOK, 2), jnp.int32),
            jax.ShapeDtypeStruct((N_TOK, 2), jnp.float32),
        ],
        scratch_shapes=[pltpu.VMEM((1, N_EXPERTS), jnp.float32)],
    )(x, Wg)


N_CH = TOK_PER_W // CHUNK


def _dispatch(x, ridx4):
    mesh = plsc.VectorSubcoreMesh(core_axis_name="c", subcore_axis_name="s")

    @functools.partial(
        pl.kernel,
        mesh=mesh,
        out_type=jax.ShapeDtypeStruct((EC_PAD, D_MODEL), jnp.float32),
        scratch_types=[
            pltpu.VMEM((CHUNK, D_MODEL), jnp.float32),
            pltpu.VMEM((CHUNK, D_MODEL), jnp.float32),
            pltpu.VMEM((N_CH, CHUNK), jnp.int32),
            pltpu.VMEM((N_CH, CHUNK), jnp.int32),
            pltpu.SemaphoreType.DMA,
            pltpu.SemaphoreType.DMA,
            pltpu.SemaphoreType.DMA,
            pltpu.SemaphoreType.DMA,
        ],
    )
    def dispatch_kernel(x_hbm, ridx4_hbm, xbuf_hbm, xv0, xv1, iv0, iv1,
                        l0, l1, s0, s1):
        wid = lax.axis_index("s") * 2 + lax.axis_index("c")
        base = wid * TOK_PER_W
        pltpu.sync_copy(ridx4_hbm.at[0, wid], iv0)
        pltpu.sync_copy(ridx4_hbm.at[1, wid], iv1)
        xvs = (xv0, xv1)
        lsems = (l0, l1)
        ssems = (s0, s1)
        loads = {}
        scats = {}
        loads[0] = pltpu.async_copy(x_hbm.at[pl.ds(base, CHUNK)], xv0, l0)
        for k in range(N_CH):
            b = k % 2
            if k + 1 < N_CH:
                nb = (k + 1) % 2
                if k - 1 >= 0:
                    scats[k - 1][0].wait()
                    scats[k - 1][1].wait()
                loads[k + 1] = pltpu.async_copy(
                    x_hbm.at[pl.ds(base + (k + 1) * CHUNK, CHUNK)],
                    xvs[nb], lsems[nb])
            loads[k].wait()
            c0 = pltpu.async_copy(xvs[b], xbuf_hbm.at[iv0.at[k]], ssems[b])
            c1 = pltpu.async_copy(xvs[b], xbuf_hbm.at[iv1.at[k]], ssems[b])
            scats[k] = (c0, c1)
        for k in (N_CH - 2, N_CH - 1):
            scats[k][0].wait()
            scats[k][1].wait()

    return dispatch_kernel(x, ridx4)


F_TILE = 1024
N_FT = D_FF // F_TILE
D_PK = D_MODEL // 2


def _ffn_body(x_ref, w1_ref, w2_ref, y_ref, acc_ref):
    f = pl.program_id(1)
    h = jnp.maximum(
        jnp.dot(x_ref[...], w1_ref[0], preferred_element_type=jnp.float32),
        0.0)
    part = jnp.dot(h, w2_ref[0], preferred_element_type=jnp.float32)

    @pl.when(f == 0)
    def _():
        acc_ref[...] = part

    @pl.when(f != 0)
    def _():
        yb = (acc_ref[...] + part).astype(jnp.bfloat16)
        hi = lax.bitcast_convert_type(yb[:, :D_PK], jnp.int16).astype(jnp.int32)
        lo = lax.bitcast_convert_type(yb[:, D_PK:], jnp.int16).astype(jnp.int32)
        y_ref[...] = (hi << 16) | (lo & 0xFFFF)


def _ffn(xbuf, W1, W2):
    return pl.pallas_call(
        _ffn_body,
        grid=(N_EXPERTS, N_FT),
        in_specs=[
            pl.BlockSpec((CAPACITY, D_MODEL), lambda e, f: (e, 0)),
            pl.BlockSpec((1, D_MODEL, F_TILE), lambda e, f: (e, 0, f)),
            pl.BlockSpec((1, F_TILE, D_MODEL), lambda e, f: (e, f, 0)),
        ],
        out_specs=pl.BlockSpec((CAPACITY, D_PK), lambda e, f: (e, 0)),
        out_shape=jax.ShapeDtypeStruct((EC_PAD, D_PK), jnp.int32),
        scratch_shapes=[pltpu.VMEM((CAPACITY, D_MODEL), jnp.float32)],
        compiler_params=pltpu.CompilerParams(
            dimension_semantics=("parallel", "arbitrary")),
    )(xbuf, W1, W2)


CG = 32
N_CHG = TOK_PER_W // CG


def _combine_gather(y32, gidx):
    mesh = plsc.VectorSubcoreMesh(core_axis_name="c", subcore_axis_name="s")

    @functools.partial(
        pl.kernel,
        mesh=mesh,
        out_type=jax.ShapeDtypeStruct((2, N_TOK, D_PK), jnp.int32),
        scratch_types=[
            pltpu.VMEM((2 * CG, D_PK), jnp.int32),
            pltpu.VMEM((2 * CG, D_PK), jnp.int32),
            pltpu.VMEM((N_CHG, 2 * CG), jnp.int32),
            pltpu.SemaphoreType.DMA,
            pltpu.SemaphoreType.DMA,
            pltpu.SemaphoreType.DMA,
            pltpu.SemaphoreType.DMA,
        ],
    )
    def gather_kernel(y_hbm, gidx_hbm, yg_hbm, yv0, yv1, iv, g0, g1, w0, w1):
        wid = lax.axis_index("s") * 2 + lax.axis_index("c")
        base = wid * TOK_PER_W
        pltpu.sync_copy(gidx_hbm.at[wid], iv)
        yvs = (yv0, yv1)
        gsems = (g0, g1)
        wsems = (w0, w1)
        gath = {}
        wr = {}
        gath[0] = pltpu.async_copy(y_hbm.at[iv.at[0]], yv0, g0)
        for k in range(N_CHG):
            b = k % 2
            if k + 1 < N_CHG:
                nb = (k + 1) % 2
                if k - 1 >= 0:
                    wr[k - 1][0].wait()
                    wr[k - 1][1].wait()
                gath[k + 1] = pltpu.async_copy(
                    y_hbm.at[iv.at[k + 1]], yvs[nb], gsems[nb])
            gath[k].wait()
            t0 = base + k * CG
            c0 = pltpu.async_copy(yvs[b].at[pl.ds(0, CG)],
                                  yg_hbm.at[0, pl.ds(t0, CG)], wsems[b])
            c1 = pltpu.async_copy(yvs[b].at[pl.ds(CG, CG)],
                                  yg_hbm.at[1, pl.ds(t0, CG)], wsems[b])
            wr[k] = (c0, c1)
        for k in (N_CHG - 2, N_CHG - 1):
            wr[k][0].wait()
            wr[k][1].wait()

    return gather_kernel(y32, gidx)


def _unpack_bf16(w):
    hi = lax.bitcast_convert_type(
        lax.shift_right_logical(w, 16).astype(jnp.int16), jnp.bfloat16)
    lo = lax.bitcast_convert_type(w.astype(jnp.int16), jnp.bfloat16)
    return jnp.concatenate(
        [hi.astype(jnp.float32), lo.astype(jnp.float32)], axis=1)


def _combine_body(yg_ref, ridx_ref, gate_ref, o_ref):
    v0 = ridx_ref[:, 0:1] != DUMMY
    v1 = ridx_ref[:, 1:2] != DUMMY
    a0 = jnp.where(v0, gate_ref[:, 0:1] * _unpack_bf16(yg_ref[0]), 0.0)
    a1 = jnp.where(v1, gate_ref[:, 1:2] * _unpack_bf16(yg_ref[1]), 0.0)
    o_ref[...] = a0 + a1


def _combine(yg, ridx, gates):
    B = ROUTER_B
    nblk = N_TOK // B
    return pl.pallas_call(
        _combine_body,
        grid=(nblk,),
        in_specs=[
            pl.BlockSpec((2, B, D_PK), lambda i: (0, i, 0)),
            pl.BlockSpec((B, 2), lambda i: (i, 0)),
            pl.BlockSpec((B, 2), lambda i: (i, 0)),
        ],
        out_specs=pl.BlockSpec((B, D_MODEL), lambda i: (i, 0)),
        out_shape=jax.ShapeDtypeStruct((N_TOK, D_MODEL), jnp.float32),
        compiler_params=pltpu.CompilerParams(
            dimension_semantics=("parallel",)),
    )(yg, ridx, gates)


def kernel(x, Wg, W1, W2):
    ridx, gates = _router(x, Wg)
    ridx_sc = ridx.T
    ridx4 = ridx_sc.reshape(2, NW, N_CH, CHUNK)
    gidx = (ridx_sc.reshape(2, NW, N_CHG, CG)
            .transpose(1, 2, 0, 3).reshape(NW, N_CHG, 2 * CG))
    xbuf = _dispatch(x, ridx4)
    y32 = _ffn(xbuf, W1, W2)
    yg32 = _combine_gather(y32, gidx)
    return _combine(yg32, ridx, gates)


# speedup vs baseline: 3.2850x; 1.0013x over previous
import functools

import jax
import jax.numpy as jnp
from jax import lax
from jax.experimental import pallas as pl
from jax.experimental.pallas import tpu as pltpu
from jax.experimental.pallas import tpu_sc as plsc

N_TOK = 8192
D_MODEL = 1024
D_FF = 2048
N_EXPERTS = 16
TOP_K = 2
CAPACITY = 1280
EC = N_EXPERTS * CAPACITY
DUMMY = EC
EC_PAD = EC + 8

ROUTER_B = 512
NW = 32
TOK_PER_W = N_TOK // NW
CHUNK = 32


def _router_body(x_ref, wg_ref, ridx_ref, gate_ref, cnt_ref):
    B = ROUTER_B
    E = N_EXPERTS
    pi = pl.program_id(0)

    @pl.when(pi == 0)
    def _():
        cnt_ref[...] = jnp.zeros_like(cnt_ref)

    logits = jnp.dot(x_ref[...], wg_ref[...],
                     preferred_element_type=jnp.float32)
    iota = lax.broadcasted_iota(jnp.int32, (B, E), 1)
    m1 = jnp.max(logits, axis=1, keepdims=True)
    i1 = jnp.min(jnp.where(logits == m1, iota, E), axis=1, keepdims=True)
    masked = jnp.where(iota == i1, -jnp.inf, logits)
    m2 = jnp.max(masked, axis=1, keepdims=True)
    i2 = jnp.min(jnp.where(masked == m2, iota, E), axis=1, keepdims=True)

    t = jnp.exp(m2 - m1)
    g0 = 1.0 / (1.0 + t)
    g1 = t / (1.0 + t)

    oh0 = (iota == i1).astype(jnp.float32)
    oh1 = (iota == i2).astype(jnp.float32)
    oh = oh0 + oh1
    tri = (lax.broadcasted_iota(jnp.int32, (B, B), 0) >
           lax.broadcasted_iota(jnp.int32, (B, B), 1)).astype(jnp.float32)
    excl = jnp.dot(tri, oh, preferred_element_type=jnp.float32)
    sx = cnt_ref[...] + excl
    pos0 = jnp.sum(sx * oh0, axis=1, keepdims=True).astype(jnp.int32)
    pos1 = jnp.sum(sx * oh1, axis=1, keepdims=True).astype(jnp.int32)
    cnt_ref[...] += jnp.sum(oh, axis=0, keepdims=True)

    r0 = jnp.where(pos0 < CAPACITY, i1 * CAPACITY + pos0, DUMMY)
    r1 = jnp.where(pos1 < CAPACITY, i2 * CAPACITY + pos1, DUMMY)
    ridx_ref[:, 0:1] = r0
    ridx_ref[:, 1:2] = r1
    gate_ref[:, 0:1] = g0
    gate_ref[:, 1:2] = g1


def _router(x, Wg):
    nblk = N_TOK // ROUTER_B
    return pl.pallas_call(
        _router_body,
        grid=(nblk,),
        in_specs=[
            pl.BlockSpec((ROUTER_B, D_MODEL), lambda i: (i, 0)),
            pl.BlockSpec((D_MODEL, N_EXPERTS), lambda i: (0, 0)),
        ],
        out_specs=[
            pl.BlockSpec((ROUTER_B, 2), lambda i: (i, 0)),
            pl.BlockSpec((ROUTER_B, 2), lambda i: (i, 0)),
        ],
        out_shape=[
            jax.ShapeDtypeStruct((N_TOK, 2), jnp.int32),
            jax.ShapeDtypeStruct((N_TOK, 2), jnp.float32),
        ],
        scratch_shapes=[pltpu.VMEM((1, N_EXPERTS), jnp.float32)],
    )(x, Wg)


N_CH = TOK_PER_W // CHUNK


def _dispatch(x, ridx4):
    mesh = plsc.VectorSubcoreMesh(core_axis_name="c", subcore_axis_name="s")

    @functools.partial(
        pl.kernel,
        mesh=mesh,
        out_type=jax.ShapeDtypeStruct((EC_PAD, D_MODEL), jnp.float32),
        scratch_types=[
            pltpu.VMEM((CHUNK, D_MODEL), jnp.float32),
            pltpu.VMEM((CHUNK, D_MODEL), jnp.float32),
            pltpu.VMEM((N_CH, CHUNK), jnp.int32),
            pltpu.VMEM((N_CH, CHUNK), jnp.int32),
            pltpu.SemaphoreType.DMA,
            pltpu.SemaphoreType.DMA,
            pltpu.SemaphoreType.DMA,
            pltpu.SemaphoreType.DMA,
        ],
    )
    def dispatch_kernel(x_hbm, ridx4_hbm, xbuf_hbm, xv0, xv1, iv0, iv1,
                        l0, l1, s0, s1):
        wid = lax.axis_index("s") * 2 + lax.axis_index("c")
        base = wid * TOK_PER_W
        pltpu.sync_copy(ridx4_hbm.at[0, wid], iv0)
        pltpu.sync_copy(ridx4_hbm.at[1, wid], iv1)
        xvs = (xv0, xv1)
        lsems = (l0, l1)
        ssems = (s0, s1)
        loads = {}
        scats = {}
        loads[0] = pltpu.async_copy(x_hbm.at[pl.ds(base, CHUNK)], xv0, l0)
        for k in range(N_CH):
            b = k % 2
            if k + 1 < N_CH:
                nb = (k + 1) % 2
                if k - 1 >= 0:
                    scats[k - 1][0].wait()
                    scats[k - 1][1].wait()
                loads[k + 1] = pltpu.async_copy(
                    x_hbm.at[pl.ds(base + (k + 1) * CHUNK, CHUNK)],
                    xvs[nb], lsems[nb])
            loads[k].wait()
            c0 = pltpu.async_copy(xvs[b], xbuf_hbm.at[iv0.at[k]], ssems[b])
            c1 = pltpu.async_copy(xvs[b], xbuf_hbm.at[iv1.at[k]], ssems[b])
            scats[k] = (c0, c1)
        for k in (N_CH - 2, N_CH - 1):
            scats[k][0].wait()
            scats[k][1].wait()

    return dispatch_kernel(x, ridx4)


F_TILE = 1024
N_FT = D_FF // F_TILE
D_PK = D_MODEL // 2


def _ffn_body(x_ref, w1_ref, w2_ref, y_ref, acc_ref):
    f = pl.program_id(1)
    h = jnp.maximum(
        jnp.dot(x_ref[...], w1_ref[0], preferred_element_type=jnp.float32),
        0.0)
    part = jnp.dot(h, w2_ref[0], preferred_element_type=jnp.float32)

    @pl.when(f == 0)
    def _():
        acc_ref[...] = part

    @pl.when(f != 0)
    def _():
        yb = (acc_ref[...] + part).astype(jnp.bfloat16)
        hi = lax.bitcast_convert_type(yb[:, :D_PK], jnp.int16).astype(jnp.int32)
        lo = lax.bitcast_convert_type(yb[:, D_PK:], jnp.int16).astype(jnp.int32)
        y_ref[...] = (hi << 16) | (lo & 0xFFFF)


def _ffn(xbuf, W1, W2):
    return pl.pallas_call(
        _ffn_body,
        grid=(N_EXPERTS, N_FT),
        in_specs=[
            pl.BlockSpec((CAPACITY, D_MODEL), lambda e, f: (e, 0)),
            pl.BlockSpec((1, D_MODEL, F_TILE), lambda e, f: (e, 0, f)),
            pl.BlockSpec((1, F_TILE, D_MODEL), lambda e, f: (e, f, 0)),
        ],
        out_specs=pl.BlockSpec((CAPACITY, D_PK), lambda e, f: (e, 0)),
        out_shape=jax.ShapeDtypeStruct((EC_PAD, D_PK), jnp.int32),
        scratch_shapes=[pltpu.VMEM((CAPACITY, D_MODEL), jnp.float32)],
        compiler_params=pltpu.CompilerParams(
            dimension_semantics=("parallel", "arbitrary")),
    )(xbuf, W1, W2)


CG = 32
N_CHG = TOK_PER_W // CG


def _combine_gather(y32, gidx):
    mesh = plsc.VectorSubcoreMesh(core_axis_name="c", subcore_axis_name="s")

    @functools.partial(
        pl.kernel,
        mesh=mesh,
        out_type=jax.ShapeDtypeStruct((2, N_TOK, D_PK), jnp.int32),
        scratch_types=[
            pltpu.VMEM((2 * CG, D_PK), jnp.int32),
            pltpu.VMEM((2 * CG, D_PK), jnp.int32),
            pltpu.VMEM((N_CHG, 2 * CG), jnp.int32),
            pltpu.SemaphoreType.DMA,
            pltpu.SemaphoreType.DMA,
            pltpu.SemaphoreType.DMA,
            pltpu.SemaphoreType.DMA,
        ],
    )
    def gather_kernel(y_hbm, gidx_hbm, yg_hbm, yv0, yv1, iv, g0, g1, w0, w1):
        wid = lax.axis_index("s") * 2 + lax.axis_index("c")
        base = wid * TOK_PER_W
        pltpu.sync_copy(gidx_hbm.at[wid], iv)
        yvs = (yv0, yv1)
        gsems = (g0, g1)
        wsems = (w0, w1)
        gath = {}
        wr = {}
        gath[0] = pltpu.async_copy(y_hbm.at[iv.at[0]], yv0, g0)
        for k in range(N_CHG):
            b = k % 2
            if k + 1 < N_CHG:
                nb = (k + 1) % 2
                if k - 1 >= 0:
                    wr[k - 1][0].wait()
                    wr[k - 1][1].wait()
                gath[k + 1] = pltpu.async_copy(
                    y_hbm.at[iv.at[k + 1]], yvs[nb], gsems[nb])
            gath[k].wait()
            t0 = base + k * CG
            c0 = pltpu.async_copy(yvs[b].at[pl.ds(0, CG)],
                                  yg_hbm.at[0, pl.ds(t0, CG)], wsems[b])
            c1 = pltpu.async_copy(yvs[b].at[pl.ds(CG, CG)],
                                  yg_hbm.at[1, pl.ds(t0, CG)], wsems[b])
            wr[k] = (c0, c1)
        for k in (N_CHG - 2, N_CHG - 1):
            wr[k][0].wait()
            wr[k][1].wait()

    return gather_kernel(y32, gidx)


def _unpack_bf16(w):
    hi = lax.bitcast_convert_type(
        lax.shift_right_logical(w, 16).astype(jnp.int16), jnp.bfloat16)
    lo = lax.bitcast_convert_type(w.astype(jnp.int16), jnp.bfloat16)
    return jnp.concatenate(
        [hi.astype(jnp.float32), lo.astype(jnp.float32)], axis=1)


def _combine_body(yg_ref, ridx_ref, gate_ref, o_ref):
    v0 = ridx_ref[:, 0:1] != DUMMY
    v1 = ridx_ref[:, 1:2] != DUMMY
    a0 = jnp.where(v0, gate_ref[:, 0:1] * _unpack_bf16(yg_ref[0]), 0.0)
    a1 = jnp.where(v1, gate_ref[:, 1:2] * _unpack_bf16(yg_ref[1]), 0.0)
    o_ref[...] = a0 + a1


def _combine(yg, ridx, gates):
    B = ROUTER_B
    nblk = N_TOK // B
    return pl.pallas_call(
        _combine_body,
        grid=(nblk,),
        in_specs=[
            pl.BlockSpec((2, B, D_PK), lambda i: (0, i, 0)),
            pl.BlockSpec((B, 2), lambda i: (i, 0)),
            pl.BlockSpec((B, 2), lambda i: (i, 0)),
        ],
        out_specs=pl.BlockSpec((B, D_MODEL), lambda i: (i, 0)),
        out_shape=jax.ShapeDtypeStruct((N_TOK, D_MODEL), jnp.float32),
        compiler_params=pltpu.CompilerParams(
            dimension_semantics=("parallel",)),
    )(yg, ridx, gates)


def kernel(x, Wg, W1, W2):
    ridx, gates = _router(x, Wg)
    ridx_sc = ridx.T
    ridx4 = ridx_sc.reshape(2, NW, N_CH, CHUNK)
    gidx = (ridx_sc.reshape(2, NW, N_CHG, CG)
            .transpose(1, 2, 0, 3).reshape(NW, N_CHG, 2 * CG))
    xbuf = _dispatch(x, ridx4)
    y32 = _ffn(xbuf, W1, W2)
    yg32 = _combine_gather(y32, gidx)
    return _combine(yg32, ridx, gates)
